# baseline (speedup 1.0000x reference)
import functools

import jax
import jax.numpy as jnp
from jax import lax
from jax.experimental import pallas as pl
from jax.experimental.pallas import tpu as pltpu
from jax.experimental.pallas import tpu_sc as plsc

N = 10000
E = 320000
D = 128
H = 64
NC = 2
NS = 16
NW = NC * NS
EPW = E // NW
K = 80
NCHUNK = EPW // K
L = 16
DW = D // 2


def _prep_body(y_ref, t_ref, x_ref, w_ref, b_ref,
               p_ref, q_ref, g_ref, h_ref, base_ref):
    x = x_ref[...]
    w = w_ref[...]
    xt = lax.dot_general(x, w[:D, :], (((1,), (0,)), ((), ())),
                         preferred_element_type=jnp.float32)
    z = lax.dot_general(x, w[D:, :], (((1,), (0,)), ((), ())),
                        preferred_element_type=jnp.float32) + b_ref[...]
    lane = lax.broadcasted_iota(jnp.int32, (N, D), 1)
    is_mu = lane < H
    ext = jnp.exp(xt) - 1.0
    ez = jnp.exp(z) - 1.0
    p_ref[...] = jnp.where(is_mu, xt, ext)
    q_ref[...] = jnp.where(is_mu, 2.0 * z, ez)
    g_ref[...] = jnp.sum(jnp.where(is_mu, xt * xt, ext - xt), axis=1,
                         keepdims=True)
    h_ref[...] = jnp.sum(jnp.where(is_mu, z * z, ez - z), axis=1,
                         keepdims=True)
    base_ref[...] = jnp.reshape(jnp.sum(jnp.abs(y_ref[...] - t_ref[...])), (1, 1))


_prep = pl.pallas_call(
    _prep_body,
    out_shape=[
        jax.ShapeDtypeStruct((N, D), jnp.float32),
        jax.ShapeDtypeStruct((N, D), jnp.float32),
        jax.ShapeDtypeStruct((N, 1), jnp.float32),
        jax.ShapeDtypeStruct((N, 1), jnp.float32),
        jax.ShapeDtypeStruct((1, 1), jnp.float32),
    ],
)


@functools.cache
def _make_edge_kernel():
    @functools.partial(
        pl.kernel,
        mesh=plsc.VectorSubcoreMesh(core_axis_name="c", subcore_axis_name="s"),
        out_type=jax.ShapeDtypeStruct((NW, L), jnp.float32),
        compiler_params=pltpu.CompilerParams(needs_layout_passes=False,
                                             use_tc_tiling_on_sc=False),
        scratch_types=[
            pltpu.VMEM((EPW,), jnp.int32),
            pltpu.VMEM((EPW,), jnp.int32),
            pltpu.VMEM((K, DW), jnp.int32),
            pltpu.VMEM((K, DW), jnp.int32),
            pltpu.VMEM((K, DW), jnp.int32),
            pltpu.VMEM((K, DW), jnp.int32),
            pltpu.VMEM((N,), jnp.float32),
            pltpu.VMEM((N,), jnp.float32),
            pltpu.VMEM((L,), jnp.float32),
            pltpu.SemaphoreType.DMA,
            pltpu.SemaphoreType.DMA,
        ],
    )
    def _edge_kernel(src_hbm, dst_hbm, p_hbm, q_hbm, g_hbm, h_hbm, out_hbm,
                     idx_s, idx_d, prow0, qrow0, prow1, qrow1,
                     g_v, h_v, accv, sem0, sem1):
        wid = lax.axis_index("s") * NC + lax.axis_index("c")
        base = wid * EPW
        pltpu.sync_copy(src_hbm.at[pl.ds(base, EPW)], idx_s)
        pltpu.sync_copy(dst_hbm.at[pl.ds(base, EPW)], idx_d)
        pltpu.sync_copy(g_hbm, g_v)
        pltpu.sync_copy(h_hbm, h_v)

        prow = (prow0, prow1)
        qrow = (qrow0, qrow1)
        sem = (sem0, sem1)

        def fire(ci, b):
            pltpu.async_copy(p_hbm.at[idx_s.at[pl.ds(ci * K, K)]], prow[b], sem[b])
            pltpu.async_copy(q_hbm.at[idx_d.at[pl.ds(ci * K, K)]], qrow[b], sem[b])

        def drain(ci, b):
            pltpu.make_async_copy(
                p_hbm.at[idx_s.at[pl.ds(ci * K, K)]], prow[b], sem[b]).wait()
            pltpu.make_async_copy(
                q_hbm.at[idx_d.at[pl.ds(ci * K, K)]], qrow[b], sem[b]).wait()

        def compute(ci, b, acc):
            off = ci * K

            def edge_body(e, a):
                for c in range(DW // L):
                    wp = prow[b][e, pl.ds(c * L, L)]
                    wq = qrow[b][e, pl.ds(c * L, L)]
                    plo = plsc.bitcast(wp << 16, jnp.float32)
                    phi = plsc.bitcast(wp, jnp.float32)
                    qlo = plsc.bitcast(wq << 16, jnp.float32)
                    qhi = plsc.bitcast(wq, jnp.float32)
                    a = a + plo * qlo + phi * qhi
                return a

            def gh_body(t, a):
                iv_s = idx_s[pl.ds(off + t * L, L)]
                iv_d = idx_d[pl.ds(off + t * L, L)]
                return (a + plsc.load_gather(g_v, [iv_s])
                        + plsc.load_gather(h_v, [iv_d]))

            acc = lax.fori_loop(0, K, edge_body, acc)
            return lax.fori_loop(0, K // L, gh_body, acc)

        fire(0, 0)

        def pair_body(i, acc):
            c0 = i * 2
            fire(c0 + 1, 1)
            drain(c0, 0)
            acc = compute(c0, 0, acc)
            fire(c0 + 2, 0)
            drain(c0 + 1, 1)
            return compute(c0 + 1, 1, acc)

        acc = lax.fori_loop(0, (NCHUNK - 1) // 2, pair_body,
                            jnp.zeros((L,), jnp.float32))
        last = NCHUNK - 1
        drain(last, 0)
        acc = compute(last, 0, acc)
        accv[...] = acc
        pltpu.sync_copy(accv, out_hbm.at[wid])

    return _edge_kernel


def _pack(t):
    return lax.bitcast_convert_type(
        t.astype(jnp.bfloat16).reshape(N, DW, 2), jnp.int32)


def kernel(y, target, x, edge_index, W_msg, b_msg):
    p, q, g, h, base = _prep(y, target, x, W_msg, b_msg.reshape(1, D))
    part = _make_edge_kernel()(edge_index[0], edge_index[1], _pack(p), _pack(q),
                               g.reshape(N), h.reshape(N))
    return base[0, 0] / N + 0.5 * jnp.sum(part) / E


# speedup vs baseline: 1.1213x; 1.1213x over previous
import functools

import jax
import jax.numpy as jnp
from jax import lax
from jax.experimental import pallas as pl
from jax.experimental.pallas import tpu as pltpu
from jax.experimental.pallas import tpu_sc as plsc

N = 10000
E = 320000
D = 128
H = 64
NC = 2
NS = 16
NW = NC * NS
EPW = E // NW
K = 80
NCHUNK = EPW // K
L = 16
NSPLIT = 2


def _prep_body(y_ref, t_ref, x_ref, w_ref, b_ref,
               p_ref, q_ref, g_ref, h_ref, base_ref):
    x = x_ref[...]
    w = w_ref[...]
    xt = lax.dot_general(x, w[:D, :], (((1,), (0,)), ((), ())),
                         preferred_element_type=jnp.float32)
    z = lax.dot_general(x, w[D:, :], (((1,), (0,)), ((), ())),
                        preferred_element_type=jnp.float32) + b_ref[...]
    lane = lax.broadcasted_iota(jnp.int32, (N, D), 1)
    is_mu = lane < H
    ext = jnp.exp(xt) - 1.0
    ez = jnp.exp(z) - 1.0
    p_ref[...] = jnp.where(is_mu, xt, ext)
    q_ref[...] = jnp.where(is_mu, 2.0 * z, ez)
    g_ref[...] = jnp.sum(jnp.where(is_mu, xt * xt, ext - xt), axis=1,
                         keepdims=True)
    h_ref[...] = jnp.sum(jnp.where(is_mu, z * z, ez - z), axis=1,
                         keepdims=True)
    base_ref[...] = jnp.reshape(jnp.sum(jnp.abs(y_ref[...] - t_ref[...])), (1, 1))


_prep = pl.pallas_call(
    _prep_body,
    out_shape=[
        jax.ShapeDtypeStruct((N, D), jnp.float32),
        jax.ShapeDtypeStruct((N, D), jnp.float32),
        jax.ShapeDtypeStruct((N, 1), jnp.float32),
        jax.ShapeDtypeStruct((N, 1), jnp.float32),
        jax.ShapeDtypeStruct((1, 1), jnp.float32),
    ],
)


@functools.cache
def _make_edge_kernel():
    @functools.partial(
        pl.kernel,
        mesh=plsc.VectorSubcoreMesh(core_axis_name="c", subcore_axis_name="s"),
        out_type=jax.ShapeDtypeStruct((NW, L), jnp.float32),
        compiler_params=pltpu.CompilerParams(needs_layout_passes=False),
        scratch_types=[
            pltpu.VMEM((EPW,), jnp.int32),
            pltpu.VMEM((EPW,), jnp.int32),
            pltpu.VMEM((K, D), jnp.float32),
            pltpu.VMEM((K, D), jnp.float32),
            pltpu.VMEM((K, D), jnp.float32),
            pltpu.VMEM((K, D), jnp.float32),
            pltpu.VMEM((N,), jnp.float32),
            pltpu.VMEM((N,), jnp.float32),
            pltpu.VMEM((L,), jnp.float32),
            pltpu.SemaphoreType.DMA,
            pltpu.SemaphoreType.DMA,
        ],
    )
    def _edge_kernel(src_hbm, dst_hbm, p_hbm, q_hbm, g_hbm, h_hbm, out_hbm,
                     idx_s, idx_d, prow0, qrow0, prow1, qrow1,
                     g_v, h_v, accv, sem0, sem1):
        wid = lax.axis_index("s") * NC + lax.axis_index("c")
        base = wid * EPW
        pltpu.sync_copy(src_hbm.at[pl.ds(base, EPW)], idx_s)
        pltpu.sync_copy(dst_hbm.at[pl.ds(base, EPW)], idx_d)
        pltpu.sync_copy(g_hbm, g_v)
        pltpu.sync_copy(h_hbm, h_v)

        prow = (prow0, prow1)
        qrow = (qrow0, qrow1)
        sem = (sem0, sem1)

        KS = K // NSPLIT

        def _legs(ci, b):
            for s in range(NSPLIT):
                o = ci * K + s * KS
                r = pl.ds(s * KS, KS)
                yield p_hbm.at[idx_s.at[pl.ds(o, KS)]], prow[b].at[r], sem[b]
                yield q_hbm.at[idx_d.at[pl.ds(o, KS)]], qrow[b].at[r], sem[b]

        def fire(ci, b):
            for src, dst, sm in _legs(ci, b):
                pltpu.async_copy(src, dst, sm)

        def drain(ci, b):
            for src, dst, sm in _legs(ci, b):
                pltpu.make_async_copy(src, dst, sm).wait()

        def compute(ci, b, acc):
            off = ci * K

            def edge_body(e, a):
                for c in range(D // L):
                    a = a + prow[b][e, pl.ds(c * L, L)] * qrow[b][e, pl.ds(c * L, L)]
                return a

            def gh_body(t, a):
                iv_s = idx_s[pl.ds(off + t * L, L)]
                iv_d = idx_d[pl.ds(off + t * L, L)]
                return (a + plsc.load_gather(g_v, [iv_s])
                        + plsc.load_gather(h_v, [iv_d]))

            acc = lax.fori_loop(0, K, edge_body, acc)
            return lax.fori_loop(0, K // L, gh_body, acc)

        fire(0, 0)

        def pair_body(i, acc):
            c0 = i * 2
            fire(c0 + 1, 1)
            drain(c0, 0)
            acc = compute(c0, 0, acc)
            fire(c0 + 2, 0)
            drain(c0 + 1, 1)
            return compute(c0 + 1, 1, acc)

        acc = lax.fori_loop(0, (NCHUNK - 1) // 2, pair_body,
                            jnp.zeros((L,), jnp.float32))
        last = NCHUNK - 1
        drain(last, 0)
        acc = compute(last, 0, acc)
        accv[...] = acc
        pltpu.sync_copy(accv, out_hbm.at[wid])

    return _edge_kernel


def kernel(y, target, x, edge_index, W_msg, b_msg):
    p, q, g, h, base = _prep(y, target, x, W_msg, b_msg.reshape(1, D))
    part = _make_edge_kernel()(edge_index[0], edge_index[1], p, q,
                               g.reshape(N), h.reshape(N))
    return base[0, 0] / N + 0.5 * jnp.sum(part) / E


# speedup vs baseline: 1.1866x; 1.0583x over previous
import functools

import jax
import jax.numpy as jnp
from jax import lax
from jax.experimental import pallas as pl
from jax.experimental.pallas import tpu as pltpu
from jax.experimental.pallas import tpu_sc as plsc

N = 10000
E = 320000
D = 128
H = 64
DW = D // 2
NC = 2
NS = 16
NW = NC * NS
EPW = E // NW
K = 80
NCHUNK = EPW // K
L = 16
RPT = (N // NS) // 8 * 8
RTAIL = N - NS * RPT


def _prep_body(y_ref, t_ref, x_ref, w_ref, b_ref,
               p_ref, q_ref, g_ref, h_ref, base_ref):
    x = x_ref[...]
    w = w_ref[...]
    xt = lax.dot_general(x, w[:D, :], (((1,), (0,)), ((), ())),
                         preferred_element_type=jnp.float32)
    z = lax.dot_general(x, w[D:, :], (((1,), (0,)), ((), ())),
                        preferred_element_type=jnp.float32) + b_ref[...]
    lane = lax.broadcasted_iota(jnp.int32, (N, D), 1)
    is_mu = lane < H
    ext = jnp.exp(xt) - 1.0
    ez = jnp.exp(z) - 1.0
    p_ref[...] = jnp.where(is_mu, xt, ext)
    q_ref[...] = jnp.where(is_mu, 2.0 * z, ez)
    g_ref[...] = jnp.sum(jnp.where(is_mu, xt * xt, ext - xt), axis=1,
                         keepdims=True)
    h_ref[...] = jnp.sum(jnp.where(is_mu, z * z, ez - z), axis=1,
                         keepdims=True)
    base_ref[...] = jnp.reshape(jnp.sum(jnp.abs(y_ref[...] - t_ref[...])), (1, 1))


_prep = pl.pallas_call(
    _prep_body,
    out_shape=[
        jax.ShapeDtypeStruct((N, D), jnp.float32),
        jax.ShapeDtypeStruct((N, D), jnp.float32),
        jax.ShapeDtypeStruct((N, 1), jnp.float32),
        jax.ShapeDtypeStruct((N, 1), jnp.float32),
        jax.ShapeDtypeStruct((1, 1), jnp.float32),
    ],
)


def _final_body(p_ref, s_ref, part_ref, out_ref):
    s = s_ref[0].astype(jnp.float32) + s_ref[1].astype(jnp.float32)
    tot = jnp.sum(p_ref[...] * s) + jnp.sum(part_ref[...])
    out_ref[...] = jnp.reshape(tot, (1, 1))


_final = pl.pallas_call(
    _final_body,
    out_shape=jax.ShapeDtypeStruct((1, 1), jnp.float32),
)


@functools.cache
def _make_edge_kernel():
    @functools.partial(
        pl.kernel,
        mesh=plsc.VectorSubcoreMesh(core_axis_name="c", subcore_axis_name="s"),
        out_type=[
            jax.ShapeDtypeStruct((NW, L), jnp.float32),
            jax.ShapeDtypeStruct((NC, N, D), jnp.bfloat16),
        ],
        compiler_params=pltpu.CompilerParams(needs_layout_passes=False,
                                             use_tc_tiling_on_sc=False),
        scratch_types=[
            pltpu.VMEM((NCHUNK, K), jnp.int32),
            pltpu.VMEM((EPW,), jnp.int32),
            pltpu.VMEM((K, DW), jnp.int32),
            pltpu.VMEM((K, DW), jnp.int32),
            pltpu.VMEM((K, D), jnp.bfloat16),
            pltpu.VMEM((K, D), jnp.bfloat16),
            pltpu.VMEM_SHARED((N, D), jnp.bfloat16),
            pltpu.VMEM((N,), jnp.float32),
            pltpu.VMEM((N,), jnp.float32),
            pltpu.VMEM((L,), jnp.float32),
            pltpu.SemaphoreType.DMA,
            pltpu.SemaphoreType.DMA,
            pltpu.SemaphoreType.DMA,
            pltpu.SemaphoreType.DMA,
        ],
    )
    def _edge_kernel(src_hbm, dst_hbm, qpk_hbm, g_hbm, h_hbm,
                     out_hbm, s_out_hbm,
                     idx_s, idx_d, qp0, qp1, qb0, qb1, s_sh,
                     g_v, h_v, accv, sg0, sg1, sw0, sw1):
        sid = lax.axis_index("s")
        cid = lax.axis_index("c")
        wid = sid * NC + cid
        base = wid * EPW
        pltpu.sync_copy(src_hbm.at[wid], idx_s)
        pltpu.sync_copy(dst_hbm.at[pl.ds(base, EPW)], idx_d)
        pltpu.sync_copy(g_hbm, g_v)
        pltpu.sync_copy(h_hbm, h_v)

        qp = (qp0, qp1)
        qb = (qb0, qb1)
        sg = (sg0, sg1)
        sw = (sw0, sw1)

        def zrow(r, _):
            for c in range(D // 32):
                qb0[r, pl.ds(c * 32, 32)] = jnp.zeros((32,), jnp.bfloat16)
            return 0

        lax.fori_loop(0, K, zrow, 0)
        t0 = sid * RPT
        for j in range(RPT // K):
            pltpu.sync_copy(qb0.at[pl.ds(0, K)], s_sh.at[pl.ds(t0 + j * K, K)])
        rem = RPT - (RPT // K) * K
        if rem:
            pltpu.sync_copy(qb0.at[pl.ds(0, rem)],
                            s_sh.at[pl.ds(t0 + (RPT // K) * K, rem)])

        @pl.when(sid == 0)
        def _zero_tail():
            pltpu.sync_copy(qb0.at[pl.ds(0, RTAIL)],
                            s_sh.at[pl.ds(NS * RPT, RTAIL)])

        plsc.subcore_barrier()

        def fire_g(ci, b):
            pltpu.async_copy(qpk_hbm.at[idx_d.at[pl.ds(ci * K, K)]],
                             qp[b], sg[b])

        def drain_g(ci, b):
            pltpu.make_async_copy(qpk_hbm.at[idx_d.at[pl.ds(ci * K, K)]],
                                  qp[b], sg[b]).wait()

        def fire_s(ci, b):
            pltpu.async_copy(qb[b], s_sh.at[idx_s.at[ci]], sw[b], add=True)

        def drain_s(ci, b):
            pltpu.make_async_copy(qb[b], s_sh.at[idx_s.at[ci]], sw[b]).wait()

        def convert(b):
            def crow(e, _):
                for c in range(DW // L):
                    w = qp[b][e, pl.ds(c * L, L)]
                    qb[b][e, pl.ds(c * 2 * L, 2 * L)] = plsc.bitcast(
                        w, jnp.bfloat16)
                return 0

            lax.fori_loop(0, K, crow, 0)

        def gh_acc(ci, acc):
            def gh_body(t, a):
                iv_s = idx_s[ci, pl.ds(t * L, L)]
                iv_d = idx_d[pl.ds(ci * K + t * L, L)]
                return (a + plsc.load_gather(g_v, [iv_s])
                        + plsc.load_gather(h_v, [iv_d]))

            return lax.fori_loop(0, K // L, gh_body, acc)

        def step(ci, b, acc):
            drain_g(ci, b)
            convert(b)
            fire_s(ci, b)
            acc = gh_acc(ci, acc)

            @pl.when(ci + 2 < NCHUNK)
            def _refire():
                fire_g(ci + 2, b)

            return acc

        fire_g(0, 0)
        fire_g(1, 1)

        def pair_body(i, acc):
            c0 = i * 2
            acc = step(c0, 0, acc)
            acc = step(c0 + 1, 1, acc)
            drain_s(c0, 0)
            drain_s(c0 + 1, 1)
            return acc

        acc = lax.fori_loop(0, NCHUNK // 2, pair_body,
                            jnp.zeros((L,), jnp.float32))
        last = NCHUNK - 1
        acc = step(last, 0, acc)
        drain_s(last, 0)
        accv[...] = acc
        pltpu.sync_copy(accv, out_hbm.at[wid])

        plsc.subcore_barrier()
        pltpu.sync_copy(s_sh.at[pl.ds(t0, RPT)],
                        s_out_hbm.at[cid, pl.ds(t0, RPT)])

        @pl.when(sid == 0)
        def _dump_tail():
            pltpu.sync_copy(s_sh.at[pl.ds(NS * RPT, RTAIL)],
                            s_out_hbm.at[cid, pl.ds(NS * RPT, RTAIL)])

    return _edge_kernel


def kernel(y, target, x, edge_index, W_msg, b_msg):
    p, q, g, h, base = _prep(y, target, x, W_msg, b_msg.reshape(1, D))
    q_packed = lax.bitcast_convert_type(
        q.astype(jnp.bfloat16).reshape(N, DW, 2), jnp.int32)
    src3d = edge_index[0].reshape(NW, NCHUNK, K)
    part, s_acc = _make_edge_kernel()(src3d, edge_index[1], q_packed,
                                      g.reshape(N), h.reshape(N))
    tot = _final(p, s_acc, part)
    return base[0, 0] / N + 0.5 * tot[0, 0] / E


# speedup vs baseline: 1.2502x; 1.0536x over previous
import functools

import jax
import jax.numpy as jnp
from jax import lax
from jax.experimental import pallas as pl
from jax.experimental.pallas import tpu as pltpu
from jax.experimental.pallas import tpu_sc as plsc

N = 10000
E = 320000
D = 128
H = 64
DW = D // 2
NC = 2
NS = 16
NW = NC * NS
EPW = E // NW
K = 80
NCHUNK = EPW // K
L = 16
RPT = (N // NS) // 8 * 8
RTAIL = N - NS * RPT


def _prep_body(y_ref, t_ref, x_ref, w_ref, b_ref,
               p_ref, q_ref, g_ref, h_ref, base_ref):
    x = x_ref[...]
    w = w_ref[...]
    xt = lax.dot_general(x, w[:D, :], (((1,), (0,)), ((), ())),
                         preferred_element_type=jnp.float32)
    z = lax.dot_general(x, w[D:, :], (((1,), (0,)), ((), ())),
                        preferred_element_type=jnp.float32) + b_ref[...]
    lane = lax.broadcasted_iota(jnp.int32, (N, D), 1)
    is_mu = lane < H
    ext = jnp.exp(xt) - 1.0
    ez = jnp.exp(z) - 1.0
    p_ref[...] = jnp.where(is_mu, xt, ext)
    q_ref[...] = jnp.where(is_mu, 2.0 * z, ez)
    g_ref[...] = jnp.sum(jnp.where(is_mu, xt * xt, ext - xt), axis=1,
                         keepdims=True)
    h_ref[...] = jnp.sum(jnp.where(is_mu, z * z, ez - z), axis=1,
                         keepdims=True)
    base_ref[...] = jnp.reshape(jnp.sum(jnp.abs(y_ref[...] - t_ref[...])), (1, 1))


_prep = pl.pallas_call(
    _prep_body,
    out_shape=[
        jax.ShapeDtypeStruct((N, D), jnp.float32),
        jax.ShapeDtypeStruct((N, D), jnp.float32),
        jax.ShapeDtypeStruct((N, 1), jnp.float32),
        jax.ShapeDtypeStruct((N, 1), jnp.float32),
        jax.ShapeDtypeStruct((1, 1), jnp.float32),
    ],
)


def _final_body(p_ref, s_ref, part_ref, base_ref, out_ref):
    s = s_ref[0].astype(jnp.float32) + s_ref[1].astype(jnp.float32)
    edge_sum = jnp.sum(p_ref[...] * s) + jnp.sum(part_ref[...])
    tot = base_ref[0, 0] / N + 0.5 * edge_sum / E
    out_ref[...] = jnp.reshape(tot, (1, 1))


_final = pl.pallas_call(
    _final_body,
    out_shape=jax.ShapeDtypeStruct((1, 1), jnp.float32),
)


@functools.cache
def _make_edge_kernel():
    @functools.partial(
        pl.kernel,
        mesh=plsc.VectorSubcoreMesh(core_axis_name="c", subcore_axis_name="s"),
        out_type=[
            jax.ShapeDtypeStruct((NW, L), jnp.float32),
            jax.ShapeDtypeStruct((NC, N, D), jnp.bfloat16),
        ],
        compiler_params=pltpu.CompilerParams(needs_layout_passes=False,
                                             use_tc_tiling_on_sc=False),
        scratch_types=[
            pltpu.VMEM((EPW,), jnp.int32),
            pltpu.VMEM((EPW,), jnp.int32),
            pltpu.VMEM((K, DW), jnp.int32),
            pltpu.VMEM((K, DW), jnp.int32),
            pltpu.VMEM((K, D), jnp.bfloat16),
            pltpu.VMEM((K, D), jnp.bfloat16),
            pltpu.VMEM_SHARED((N, D), jnp.bfloat16),
            pltpu.VMEM((N,), jnp.float32),
            pltpu.VMEM((N,), jnp.float32),
            pltpu.VMEM((L,), jnp.float32),
            pltpu.SemaphoreType.DMA,
            pltpu.SemaphoreType.DMA,
            pltpu.SemaphoreType.DMA,
            pltpu.SemaphoreType.DMA,
        ],
    )
    def _edge_kernel(ei_hbm, qpk_hbm, g_hbm, h_hbm,
                     out_hbm, s_out_hbm,
                     idx_s, idx_d, qp0, qp1, qb0, qb1, s_sh,
                     g_v, h_v, accv, sg0, sg1, sw0, sw1):
        sid = lax.axis_index("s")
        cid = lax.axis_index("c")
        wid = sid * NC + cid
        base = wid * EPW
        pltpu.sync_copy(ei_hbm.at[0, pl.ds(base, EPW)], idx_s)
        pltpu.sync_copy(ei_hbm.at[1, pl.ds(base, EPW)], idx_d)
        pltpu.sync_copy(g_hbm, g_v)
        pltpu.sync_copy(h_hbm, h_v)

        qp = (qp0, qp1)
        qb = (qb0, qb1)
        sg = (sg0, sg1)
        sw = (sw0, sw1)

        def zrow(r, _):
            for c in range(D // 32):
                qb0[r, pl.ds(c * 32, 32)] = jnp.zeros((32,), jnp.bfloat16)
            return 0

        lax.fori_loop(0, K, zrow, 0)
        t0 = sid * RPT
        for j in range(RPT // K):
            pltpu.sync_copy(qb0.at[pl.ds(0, K)], s_sh.at[pl.ds(t0 + j * K, K)])
        rem = RPT - (RPT // K) * K
        if rem:
            pltpu.sync_copy(qb0.at[pl.ds(0, rem)],
                            s_sh.at[pl.ds(t0 + (RPT // K) * K, rem)])

        @pl.when(sid == 0)
        def _zero_tail():
            pltpu.sync_copy(qb0.at[pl.ds(0, RTAIL)],
                            s_sh.at[pl.ds(NS * RPT, RTAIL)])

        plsc.subcore_barrier()

        def fire_g(ci, b):
            pltpu.async_copy(qpk_hbm.at[idx_d.at[pl.ds(ci * K, K)]],
                             qp[b], sg[b])

        def drain_g(ci, b):
            pltpu.make_async_copy(qpk_hbm.at[idx_d.at[pl.ds(ci * K, K)]],
                                  qp[b], sg[b]).wait()

        def fire_s(ci, b):
            pltpu.async_copy(qb[b], s_sh.at[idx_s.at[pl.ds(ci * K, K)]],
                             sw[b], add=True)

        def drain_s(ci, b):
            pltpu.make_async_copy(qb[b], s_sh.at[idx_s.at[pl.ds(ci * K, K)]],
                                  sw[b]).wait()

        def convert(b):
            def crow(e, _):
                for c in range(DW // L):
                    w = qp[b][e, pl.ds(c * L, L)]
                    qb[b][e, pl.ds(c * 2 * L, 2 * L)] = plsc.bitcast(
                        w, jnp.bfloat16)
                return 0

            lax.fori_loop(0, K, crow, 0)

        def gh_acc(ci, acc):
            def gh_body(t, a):
                iv_s = idx_s[pl.ds(ci * K + t * L, L)]
                iv_d = idx_d[pl.ds(ci * K + t * L, L)]
                return (a + plsc.load_gather(g_v, [iv_s])
                        + plsc.load_gather(h_v, [iv_d]))

            return lax.fori_loop(0, K // L, gh_body, acc)

        def step(ci, b, acc):
            drain_g(ci, b)
            convert(b)
            fire_s(ci, b)
            acc = gh_acc(ci, acc)

            @pl.when(ci + 2 < NCHUNK)
            def _refire():
                fire_g(ci + 2, b)

            return acc

        fire_g(0, 0)
        fire_g(1, 1)

        def pair_body(i, acc):
            c0 = i * 2
            acc = step(c0, 0, acc)
            acc = step(c0 + 1, 1, acc)
            drain_s(c0, 0)
            drain_s(c0 + 1, 1)
            return acc

        acc = lax.fori_loop(0, NCHUNK // 2, pair_body,
                            jnp.zeros((L,), jnp.float32))
        last = NCHUNK - 1
        acc = step(last, 0, acc)
        drain_s(last, 0)
        accv[...] = acc
        pltpu.sync_copy(accv, out_hbm.at[wid])

        plsc.subcore_barrier()
        pltpu.sync_copy(s_sh.at[pl.ds(t0, RPT)],
                        s_out_hbm.at[cid, pl.ds(t0, RPT)])

        @pl.when(sid == 0)
        def _dump_tail():
            pltpu.sync_copy(s_sh.at[pl.ds(NS * RPT, RTAIL)],
                            s_out_hbm.at[cid, pl.ds(NS * RPT, RTAIL)])

    return _edge_kernel


def kernel(y, target, x, edge_index, W_msg, b_msg):
    p, q, g, h, base = _prep(y, target, x, W_msg, b_msg.reshape(1, D))
    q_packed = lax.bitcast_convert_type(
        q.astype(jnp.bfloat16).reshape(N, DW, 2), jnp.int32)
    part, s_acc = _make_edge_kernel()(edge_index, q_packed,
                                      g.reshape(N), h.reshape(N))
    tot = _final(p, s_acc, part, base)
    return tot[0, 0]


# speedup vs baseline: 1.2921x; 1.0335x over previous
import functools

import jax
import jax.numpy as jnp
from jax import lax
from jax.experimental import pallas as pl
from jax.experimental.pallas import tpu as pltpu
from jax.experimental.pallas import tpu_sc as plsc

N = 10000
E = 320000
D = 128
H = 64
DW = D // 2
NC = 2
NS = 16
NW = NC * NS
EPW = E // NW
K = 80
NCHUNK = EPW // K
L = 16
RPT = (N // NS) // 8 * 8
RTAIL = N - NS * RPT


def _bf16_bits(v):
    u = lax.bitcast_convert_type(v, jnp.uint32)
    return (u + jnp.uint32(0x8000)) >> 16


def _prep_body(y_ref, t_ref, x_ref, w_ref, b_ref,
               p_ref, q_ref, gh_ref, base_ref):
    x = x_ref[...]
    w = w_ref[...]
    xt = lax.dot_general(x, w[:D, :], (((1,), (0,)), ((), ())),
                         preferred_element_type=jnp.float32)
    z = lax.dot_general(x, w[D:, :], (((1,), (0,)), ((), ())),
                        preferred_element_type=jnp.float32) + b_ref[...]
    lane = lax.broadcasted_iota(jnp.int32, (N, D), 1)
    is_mu = lane < H
    ext = jnp.exp(xt) - 1.0
    ez = jnp.exp(z) - 1.0
    p_ref[...] = jnp.where(is_mu, xt, ext)
    q_ref[...] = jnp.where(is_mu, 2.0 * z, ez)
    g = jnp.sum(jnp.where(is_mu, xt * xt, ext - xt), axis=1, keepdims=True)
    h = jnp.sum(jnp.where(is_mu, z * z, ez - z), axis=1, keepdims=True)
    gh_ref[...] = lax.bitcast_convert_type(
        _bf16_bits(g) | (_bf16_bits(h) << 16), jnp.int32)
    base_ref[...] = jnp.reshape(jnp.sum(jnp.abs(y_ref[...] - t_ref[...])), (1, 1))


_prep = pl.pallas_call(
    _prep_body,
    out_shape=[
        jax.ShapeDtypeStruct((N, D), jnp.float32),
        jax.ShapeDtypeStruct((N, D), jnp.float32),
        jax.ShapeDtypeStruct((N, 1), jnp.int32),
        jax.ShapeDtypeStruct((1, 1), jnp.float32),
    ],
)


def _final_body(p_ref, s_ref, part_ref, base_ref, out_ref):
    s = s_ref[0].astype(jnp.float32) + s_ref[1].astype(jnp.float32)
    edge_sum = jnp.sum(p_ref[...] * s) + jnp.sum(part_ref[...])
    tot = base_ref[0, 0] / N + 0.5 * edge_sum / E
    out_ref[...] = jnp.reshape(tot, (1, 1))


_final = pl.pallas_call(
    _final_body,
    out_shape=jax.ShapeDtypeStruct((1, 1), jnp.float32),
)


@functools.cache
def _make_edge_kernel():
    @functools.partial(
        pl.kernel,
        mesh=plsc.VectorSubcoreMesh(core_axis_name="c", subcore_axis_name="s"),
        out_type=[
            jax.ShapeDtypeStruct((NW, L), jnp.float32),
            jax.ShapeDtypeStruct((NC, N, D), jnp.bfloat16),
        ],
        compiler_params=pltpu.CompilerParams(needs_layout_passes=False,
                                             use_tc_tiling_on_sc=False),
        scratch_types=[
            pltpu.VMEM((EPW,), jnp.int32),
            pltpu.VMEM((EPW,), jnp.int32),
            pltpu.VMEM((K, DW), jnp.int32),
            pltpu.VMEM((K, DW), jnp.int32),
            pltpu.VMEM((K, D), jnp.bfloat16),
            pltpu.VMEM((K, D), jnp.bfloat16),
            pltpu.VMEM_SHARED((N, D), jnp.bfloat16),
            pltpu.VMEM((N,), jnp.int32),
            pltpu.VMEM((L,), jnp.float32),
            pltpu.SemaphoreType.DMA,
            pltpu.SemaphoreType.DMA,
            pltpu.SemaphoreType.DMA,
            pltpu.SemaphoreType.DMA,
        ],
    )
    def _edge_kernel(ei_hbm, qpk_hbm, gh_hbm,
                     out_hbm, s_out_hbm,
                     idx_s, idx_d, qp0, qp1, qb0, qb1, s_sh,
                     gh_v, accv, sg0, sg1, sw0, sw1):
        sid = lax.axis_index("s")
        cid = lax.axis_index("c")
        wid = sid * NC + cid
        base = wid * EPW
        pltpu.sync_copy(ei_hbm.at[0, pl.ds(base, EPW)], idx_s)
        pltpu.sync_copy(ei_hbm.at[1, pl.ds(base, EPW)], idx_d)
        pltpu.sync_copy(gh_hbm, gh_v)

        qp = (qp0, qp1)
        qb = (qb0, qb1)
        sg = (sg0, sg1)
        sw = (sw0, sw1)

        def zrow(r, _):
            for c in range(D // 32):
                qb0[r, pl.ds(c * 32, 32)] = jnp.zeros((32,), jnp.bfloat16)
            return 0

        lax.fori_loop(0, K, zrow, 0)
        t0 = sid * RPT
        for j in range(RPT // K):
            pltpu.sync_copy(qb0.at[pl.ds(0, K)], s_sh.at[pl.ds(t0 + j * K, K)])
        rem = RPT - (RPT // K) * K
        if rem:
            pltpu.sync_copy(qb0.at[pl.ds(0, rem)],
                            s_sh.at[pl.ds(t0 + (RPT // K) * K, rem)])

        @pl.when(sid == 0)
        def _zero_tail():
            pltpu.sync_copy(qb0.at[pl.ds(0, RTAIL)],
                            s_sh.at[pl.ds(NS * RPT, RTAIL)])

        plsc.subcore_barrier()

        def fire_g(ci, b):
            pltpu.async_copy(qpk_hbm.at[idx_d.at[pl.ds(ci * K, K)]],
                             qp[b], sg[b])

        def drain_g(ci, b):
            pltpu.make_async_copy(qpk_hbm.at[idx_d.at[pl.ds(ci * K, K)]],
                                  qp[b], sg[b]).wait()

        def fire_s(ci, b):
            pltpu.async_copy(qb[b], s_sh.at[idx_s.at[pl.ds(ci * K, K)]],
                             sw[b], add=True)

        def drain_s(ci, b):
            pltpu.make_async_copy(qb[b], s_sh.at[idx_s.at[pl.ds(ci * K, K)]],
                                  sw[b]).wait()

        def convert(b):
            def crow(e, _):
                for c in range(DW // L):
                    w = qp[b][e, pl.ds(c * L, L)]
                    qb[b][e, pl.ds(c * 2 * L, 2 * L)] = plsc.bitcast(
                        w, jnp.bfloat16)
                return 0

            lax.fori_loop(0, K, crow, 0)

        def gh_acc(ci, acc):
            def gh_body(t, a):
                iv_s = idx_s[pl.ds(ci * K + t * L, L)]
                iv_d = idx_d[pl.ds(ci * K + t * L, L)]
                w_s = plsc.load_gather(gh_v, [iv_s])
                w_d = plsc.load_gather(gh_v, [iv_d])
                g_s = plsc.bitcast(w_s << 16, jnp.float32)
                h_d = plsc.bitcast(w_d & jnp.int32(-65536), jnp.float32)
                return a + g_s + h_d

            return lax.fori_loop(0, K // L, gh_body, acc)

        def step(ci, b, acc):
            drain_g(ci, b)
            convert(b)
            fire_s(ci, b)
            acc = gh_acc(ci, acc)

            @pl.when(ci + 2 < NCHUNK)
            def _refire():
                fire_g(ci + 2, b)

            return acc

        fire_g(0, 0)
        fire_g(1, 1)

        def pair_body(i, acc):
            c0 = i * 2
            acc = step(c0, 0, acc)
            acc = step(c0 + 1, 1, acc)
            drain_s(c0, 0)
            drain_s(c0 + 1, 1)
            return acc

        acc = lax.fori_loop(0, NCHUNK // 2, pair_body,
                            jnp.zeros((L,), jnp.float32))
        last = NCHUNK - 1
        acc = step(last, 0, acc)
        drain_s(last, 0)
        accv[...] = acc
        pltpu.sync_copy(accv, out_hbm.at[wid])

        plsc.subcore_barrier()
        pltpu.sync_copy(s_sh.at[pl.ds(t0, RPT)],
                        s_out_hbm.at[cid, pl.ds(t0, RPT)])

        @pl.when(sid == 0)
        def _dump_tail():
            pltpu.sync_copy(s_sh.at[pl.ds(NS * RPT, RTAIL)],
                            s_out_hbm.at[cid, pl.ds(NS * RPT, RTAIL)])

    return _edge_kernel


def kernel(y, target, x, edge_index, W_msg, b_msg):
    p, q, gh, base = _prep(y, target, x, W_msg, b_msg.reshape(1, D))
    q_packed = lax.bitcast_convert_type(
        q.astype(jnp.bfloat16).reshape(N, DW, 2), jnp.int32)
    part, s_acc = _make_edge_kernel()(edge_index, q_packed, gh.reshape(N))
    tot = _final(p, s_acc, part, base)
    return tot[0, 0]
